# baseline (speedup 1.0000x reference)
import functools

import jax
import jax.numpy as jnp
from jax import lax
from jax.experimental import pallas as pl
from jax.experimental.pallas import tpu as pltpu
from jax.experimental.pallas import tpu_sc as plsc

_NUM_CLASSES = 100000
_EMBED_DIM = 64
_BATCH = 16384

_NC = 2
_NS = 16
_NW = _NC * _NS
_L = 16
_FEATS_PER_W = _EMBED_DIM // _NW
_CHUNK = 4096


def _center_loss_partials(emb_t, labels, cent_t):
  mesh = plsc.VectorSubcoreMesh(core_axis_name="c", subcore_axis_name="s")

  @functools.partial(
      pl.kernel,
      mesh=mesh,
      out_type=jax.ShapeDtypeStruct((_NW, _L), jnp.float32),
      compiler_params=pltpu.CompilerParams(use_tc_tiling_on_sc=True,
                                           needs_layout_passes=False),
      scratch_types=[
          pltpu.VMEM((_NUM_CLASSES,), jnp.float32),
          pltpu.VMEM((2, _CHUNK), jnp.int32),
          pltpu.VMEM((2, _CHUNK), jnp.float32),
          pltpu.VMEM((_L,), jnp.float32),
          pltpu.SemaphoreType.DMA,
          pltpu.SemaphoreType.DMA,
      ],
  )
  def body(emb_hbm, lab_hbm, cent_hbm, out_hbm, crow_v, lab_v, erow_v, acc_v,
           sem_row, sem_chunk):
    wid = lax.axis_index("s") * _NC + lax.axis_index("c")
    n_chunks = _BATCH // _CHUNK
    n_items = _FEATS_PER_W * n_chunks
    unroll = 8

    def start_chunk(c, chunk, buf):
      base = chunk * _CHUNK
      pltpu.async_copy(lab_hbm.at[pl.ds(base, _CHUNK)], lab_v.at[buf],
                       sem_chunk)
      pltpu.async_copy(emb_hbm.at[c, pl.ds(base, _CHUNK)], erow_v.at[buf],
                       sem_chunk)

    def wait_chunk(buf):
      pltpu.make_async_copy(lab_hbm.at[pl.ds(0, _CHUNK)], lab_v.at[buf],
                            sem_chunk).wait()
      pltpu.make_async_copy(emb_hbm.at[0, pl.ds(0, _CHUNK)], erow_v.at[buf],
                            sem_chunk).wait()

    start_chunk(wid * _FEATS_PER_W, 0, 0)

    def feat_body(f, acc):
      c = wid * _FEATS_PER_W + f
      pltpu.sync_copy(cent_hbm.at[c], crow_v)

      def chunk_body(chunk, acc):
        item = f * n_chunks + chunk
        buf = lax.rem(item, 2)
        wait_chunk(buf)
        nxt = jnp.minimum(item + 1, n_items - 1)
        start_chunk(wid * _FEATS_PER_W + nxt // n_chunks,
                    lax.rem(nxt, n_chunks), 1 - buf)

        def iter_body(j, accs):
          base = j * (_L * unroll)
          new = list(accs)
          for u in range(unroll):
            lv = lab_v[buf, pl.ds(base + u * _L, _L)]
            g = plsc.load_gather(crow_v, [lv])
            e = erow_v[buf, pl.ds(base + u * _L, _L)]
            d = e - g
            new[u % 4] = new[u % 4] + d * d
          return tuple(new)

        return lax.fori_loop(0, _CHUNK // (_L * unroll), iter_body, acc)

      return lax.fori_loop(0, n_chunks, chunk_body, acc)

    acc = lax.fori_loop(
        0, _FEATS_PER_W, feat_body,
        tuple(jnp.zeros((_L,), jnp.float32) for _ in range(4)))

    wait_chunk(n_items % 2)
    total = (acc[0] + acc[1]) + (acc[2] + acc[3])
    acc_v[...] = total
    pltpu.sync_copy(acc_v, out_hbm.at[wid])

  return body(emb_t, labels, cent_t)


def kernel(embeddings, labels, centers):
  partials = _center_loss_partials(embeddings.T, labels.astype(jnp.int32),
                                   centers.T)
  return jnp.sum(partials) / _BATCH


# speedup vs baseline: 1.0817x; 1.0817x over previous
import functools

import jax
import jax.numpy as jnp
from jax import lax
from jax.experimental import pallas as pl
from jax.experimental.pallas import tpu as pltpu
from jax.experimental.pallas import tpu_sc as plsc

_NUM_CLASSES = 100000
_EMBED_DIM = 64
_BATCH = 16384

_NC = 2
_NS = 16
_NW = _NC * _NS
_L = 16
_FEATS_PER_W = _EMBED_DIM // _NW
_CHUNK = 2048
_NBUF = 4
_UNROLL = 8


def _center_loss_partials(emb_t, labels, cent_t):
  mesh = plsc.VectorSubcoreMesh(core_axis_name="c", subcore_axis_name="s")
  n_chunks = _BATCH // _CHUNK
  n_items = _FEATS_PER_W * n_chunks

  @functools.partial(
      pl.kernel,
      mesh=mesh,
      out_type=jax.ShapeDtypeStruct((_NW, _L), jnp.float32),
      compiler_params=pltpu.CompilerParams(use_tc_tiling_on_sc=True,
                                           needs_layout_passes=False),
      scratch_types=[
          pltpu.VMEM((_NUM_CLASSES,), jnp.float32),
          pltpu.VMEM((_BATCH,), jnp.int32),
          pltpu.VMEM((_NBUF, _CHUNK), jnp.float32),
          pltpu.VMEM((_L,), jnp.float32),
          pltpu.SemaphoreType.DMA,
      ],
  )
  def body(emb_hbm, lab_hbm, cent_hbm, out_hbm, crow_v, lab_v, erow_v, acc_v,
           sem_chunk):
    wid = lax.axis_index("s") * _NC + lax.axis_index("c")
    c0 = wid * _FEATS_PER_W

    def start_item(item, buf):
      f = item // n_chunks
      base = lax.rem(item, n_chunks) * _CHUNK
      pltpu.async_copy(emb_hbm.at[c0 + f, pl.ds(base, _CHUNK)],
                       erow_v.at[buf], sem_chunk)

    def wait_item():
      pltpu.make_async_copy(emb_hbm.at[0, pl.ds(0, _CHUNK)], erow_v.at[0],
                            sem_chunk).wait()

    pltpu.sync_copy(lab_hbm, lab_v)
    for p in range(_NBUF - 1):
      start_item(p, p)

    def feat_body(f, acc):
      pltpu.sync_copy(cent_hbm.at[c0 + f], crow_v)

      def chunk_body(chunk, acc):
        item = f * n_chunks + chunk
        buf = lax.rem(item, _NBUF)
        wait_item()
        start_item(jnp.minimum(item + _NBUF - 1, n_items - 1),
                   lax.rem(item + _NBUF - 1, _NBUF))
        lab_base = lax.rem(item, n_chunks) * _CHUNK

        def iter_body(j, accs):
          base = j * (_L * _UNROLL)
          new = list(accs)
          for u in range(_UNROLL):
            lv = lab_v[pl.ds(lab_base + base + u * _L, _L)]
            g = plsc.load_gather(crow_v, [lv])
            e = erow_v[buf, pl.ds(base + u * _L, _L)]
            d = e - g
            new[u % 4] = new[u % 4] + d * d
          return tuple(new)

        return lax.fori_loop(0, _CHUNK // (_L * _UNROLL), iter_body, acc)

      return lax.fori_loop(0, n_chunks, chunk_body, acc)

    acc = lax.fori_loop(
        0, _FEATS_PER_W, feat_body,
        tuple(jnp.zeros((_L,), jnp.float32) for _ in range(4)))

    for _ in range(_NBUF - 1):
      wait_item()
    total = (acc[0] + acc[1]) + (acc[2] + acc[3])
    acc_v[...] = total
    pltpu.sync_copy(acc_v, out_hbm.at[wid])

  return body(emb_t, labels, cent_t)


def kernel(embeddings, labels, centers):
  partials = _center_loss_partials(embeddings.T, labels.astype(jnp.int32),
                                   centers.T)
  return jnp.sum(partials) / _BATCH
